# baseline (speedup 1.0000x reference)
import dataclasses

import jax
import jax.numpy as jnp
from jax import lax
from jax.experimental import pallas as pl
from jax.experimental.pallas import tpu as pltpu
from jax.experimental.pallas import tpu_sc as plsc

NC = 2
NS = 16
NW = NC * NS
L = 16

B = 16384
D = 32
BPW = B // NW
NCHUNK = 4
CHUNK = BPW // NCHUNK


def _body(uid_hbm, mid_hbm, ut_hbm, mt_hbm, w_hbm, b_hbm, out_hbm,
          uidx, midx, urows, mrows, outv, w_v, b_v, sem):
    wid = lax.axis_index("s") * NC + lax.axis_index("c")
    base = wid * BPW

    pltpu.sync_copy(uid_hbm.at[wid], uidx)
    pltpu.sync_copy(mid_hbm.at[wid], midx)

    copies = []
    for j in range(NCHUNK):
        copies.append(pltpu.async_copy(
            ut_hbm.at[uidx.at[j]], urows.at[pl.ds(j * CHUNK, CHUNK)], sem))
        copies.append(pltpu.async_copy(
            mt_hbm.at[midx.at[j]], mrows.at[pl.ds(j * CHUNK, CHUNK)], sem))

    pltpu.sync_copy(w_hbm, w_v)
    pltpu.sync_copy(b_hbm, b_v)

    for c in copies:
        c.wait()

    iota = lax.iota(jnp.int32, L)
    bvec = b_v[...]

    @pl.loop(0, BPW // L)
    def _(g):
        rows = g * L + iota
        acc = jnp.zeros((L,), jnp.float32)
        for d in range(D):
            cols = jnp.full((L,), d, jnp.int32)
            gu = plsc.load_gather(urows, [rows, cols])
            gm = plsc.load_gather(mrows, [rows, cols])
            acc = acc + gu * gm * w_v[d]
        outv[pl.ds(g * L, L)] = acc + bvec

    pltpu.sync_copy(outv, out_hbm.at[pl.ds(base, BPW)])


def kernel(user_id, movie_id, user_table, movie_table, fc_w, fc_b):
    uid = user_id.astype(jnp.int32).reshape(NW, NCHUNK, CHUNK)
    mid = movie_id.astype(jnp.int32).reshape(NW, NCHUNK, CHUNK)
    wsp = jnp.broadcast_to(fc_w.reshape(D, 1), (D, L))
    b16 = jnp.broadcast_to(fc_b, (L,))

    cp = pltpu.CompilerParams(
        needs_layout_passes=False, use_tc_tiling_on_sc=False)

    run = pl.kernel(
        _body,
        out_type=jax.ShapeDtypeStruct((B,), jnp.float32),
        mesh=plsc.VectorSubcoreMesh(core_axis_name="c", subcore_axis_name="s"),
        compiler_params=cp,
        scratch_types=[
            pltpu.VMEM((NCHUNK, CHUNK), jnp.int32),
            pltpu.VMEM((NCHUNK, CHUNK), jnp.int32),
            pltpu.VMEM((BPW, D), jnp.float32),
            pltpu.VMEM((BPW, D), jnp.float32),
            pltpu.VMEM((BPW,), jnp.float32),
            pltpu.VMEM((D, L), jnp.float32),
            pltpu.VMEM((L,), jnp.float32),
            pltpu.SemaphoreType.DMA,
        ],
    )
    return run(uid, mid, user_table, movie_table, wsp, b16)
